# Initial kernel scaffold; baseline (speedup 1.0000x reference)
import jax
import jax.numpy as jnp
from jax.experimental import pallas as pl


def kernel(X, table):
    raise NotImplementedError("write your pallas kernel here")


# speedup vs baseline: 1.2430x; 1.2430x over previous
import functools

import jax
import jax.numpy as jnp
from jax import lax
from jax.experimental import pallas as pl
from jax.experimental.pallas import tpu as pltpu
from jax.experimental.pallas import tpu_sc as plsc

B = 16384
F = 26
V = 1000000

NC = 2
NS = 16
NW = NC * NS
RPW = B // NW
CHW = 128
CH = RPW // CHW
L = 16


def _build():
    mesh = plsc.VectorSubcoreMesh(core_axis_name="c", subcore_axis_name="s")

    @functools.partial(
        pl.kernel,
        mesh=mesh,
        out_type=jax.ShapeDtypeStruct((NW, CH, CHW), jnp.float32),
        scratch_types=[
            pltpu.VMEM((F * CH, CHW), jnp.int32),
            pltpu.VMEM((CH, CHW), jnp.float32),
            pltpu.VMEM((CH, CHW), jnp.float32),
            pltpu.SemaphoreType.DMA,
        ],
    )
    def k(xt_hbm, tab_hbm, out_hbm, idx_v, col_v, acc_v, sem):
        cid = lax.axis_index("c")
        sid = lax.axis_index("s")
        wid = sid * NC + cid

        pltpu.sync_copy(xt_hbm.at[wid], idx_v)

        zero = jnp.zeros((L,), jnp.float32)
        for j in range(CH):
            for t in range(CHW // L):
                acc_v[j, pl.ds(t * L, L)] = zero

        def body(f, carry):
            cps = [
                pltpu.async_copy(
                    tab_hbm.at[idx_v.at[f * CH + j]], col_v.at[j], sem
                )
                for j in range(CH)
            ]
            for cp in cps:
                cp.wait()
            for j in range(CH):
                for t in range(CHW // L):
                    sl = pl.ds(t * L, L)
                    acc_v[j, sl] = acc_v[j, sl] + col_v[j, sl]
            return carry

        lax.fori_loop(0, F, body, 0)

        pltpu.sync_copy(acc_v, out_hbm.at[wid])

    return k


_kernel = _build()


def kernel(X, table):
    xt = X.reshape(NW, CH, CHW, F).transpose(0, 3, 1, 2).reshape(NW, F * CH, CHW)
    out = _kernel(xt, table.reshape(V))
    return out.reshape(B, 1)


# speedup vs baseline: 1.4564x; 1.1717x over previous
import functools

import jax
import jax.numpy as jnp
from jax import lax
from jax.experimental import pallas as pl
from jax.experimental.pallas import tpu as pltpu
from jax.experimental.pallas import tpu_sc as plsc

B = 16384
F = 26
V = 1000000

NC = 2
NS = 16
NW = NC * NS
RPW = B // NW
CHW = 128
CH = RPW // CHW
L = 16
ROWS = F * CH
GRP = 8
NG = ROWS // GRP


def _build():
    mesh = plsc.VectorSubcoreMesh(core_axis_name="c", subcore_axis_name="s")

    @functools.partial(
        pl.kernel,
        mesh=mesh,
        out_type=jax.ShapeDtypeStruct((NW, CH, CHW), jnp.float32),
        scratch_types=[
            pltpu.VMEM((ROWS, CHW), jnp.int32),
            pltpu.VMEM((ROWS, CHW), jnp.float32),
            pltpu.VMEM((CH, CHW), jnp.float32),
            pltpu.SemaphoreType.DMA((NG,)),
        ],
    )
    def k(xt_hbm, tab_hbm, out_hbm, idx_v, col_v, res_v, sems):
        cid = lax.axis_index("c")
        sid = lax.axis_index("s")
        wid = sid * NC + cid

        pltpu.sync_copy(xt_hbm.at[wid], idx_v)

        def fire(g, carry):
            for i in range(GRP):
                r = g * GRP + i
                pltpu.async_copy(tab_hbm.at[idx_v.at[r]], col_v.at[r], sems.at[g])
            return carry

        lax.fori_loop(0, NG, fire, 0)

        acc = [jnp.zeros((L,), jnp.float32) for _ in range(CH * (CHW // L))]
        for g in range(NG):
            for i in range(GRP):
                r = g * GRP + i
                pltpu.make_async_copy(
                    tab_hbm.at[idx_v.at[r]], col_v.at[r], sems.at[g]
                ).wait()
            for r in range(g * GRP, (g + 1) * GRP):
                j = r % CH
                for t in range(CHW // L):
                    acc[j * (CHW // L) + t] = (
                        acc[j * (CHW // L) + t] + col_v[r, pl.ds(t * L, L)]
                    )

        for j in range(CH):
            for t in range(CHW // L):
                res_v[j, pl.ds(t * L, L)] = acc[j * (CHW // L) + t]
        pltpu.sync_copy(res_v, out_hbm.at[wid])

    return k


_kernel = _build()


def kernel(X, table):
    xt = X.reshape(NW, CH, CHW, F).transpose(0, 3, 1, 2).reshape(NW, ROWS, CHW)
    out = _kernel(xt, table.reshape(V))
    return out.reshape(B, 1)


# speedup vs baseline: 2.9678x; 2.0377x over previous
import functools

import jax
import jax.numpy as jnp
from jax import lax
from jax.experimental import pallas as pl
from jax.experimental.pallas import tpu as pltpu
from jax.experimental.pallas import tpu_sc as plsc

B = 16384
F = 26
V = 1000000

NC = 2
NS = 16
NW = NC * NS
RPW = B // NW
CHW = 128
CH = RPW // CHW
L = 16
ROWS = F * CH
GRP = 8
NG = ROWS // GRP


def _build():
    mesh = plsc.VectorSubcoreMesh(core_axis_name="c", subcore_axis_name="s")

    @functools.partial(
        pl.kernel,
        mesh=mesh,
        out_type=jax.ShapeDtypeStruct((NW, CH, CHW), jnp.float32),
        scratch_types=[
            pltpu.VMEM((ROWS, CHW), jnp.int32),
            pltpu.VMEM((ROWS, CHW), jnp.float32),
            pltpu.VMEM((CH, CHW), jnp.float32),
            pltpu.SemaphoreType.DMA((NG,)),
        ],
    )
    def k(xt_hbm, tab_hbm, out_hbm, idx_v, col_v, res_v, sems):
        cid = lax.axis_index("c")
        sid = lax.axis_index("s")
        wid = sid * NC + cid

        pltpu.sync_copy(xt_hbm.at[wid], idx_v)

        def fire(g, carry):
            for i in range(GRP):
                r = g * GRP + i
                pltpu.async_copy(
                    tab_hbm.at[0].at[idx_v.at[r]], col_v.at[r], sems.at[g]
                )
            return carry

        lax.fori_loop(0, NG, fire, 0)

        acc = [jnp.zeros((L,), jnp.float32) for _ in range(CH * (CHW // L))]
        for g in range(NG):
            for i in range(GRP):
                r = g * GRP + i
                pltpu.make_async_copy(
                    tab_hbm.at[0].at[idx_v.at[r]], col_v.at[r], sems.at[g]
                ).wait()
            for r in range(g * GRP, (g + 1) * GRP):
                j = r % CH
                for t in range(CHW // L):
                    acc[j * (CHW // L) + t] = (
                        acc[j * (CHW // L) + t] + col_v[r, pl.ds(t * L, L)]
                    )

        for j in range(CH):
            for t in range(CHW // L):
                res_v[j, pl.ds(t * L, L)] = acc[j * (CHW // L) + t]
        pltpu.sync_copy(res_v, out_hbm.at[wid])

    return k


_kernel = _build()


def kernel(X, table):
    xt = X.reshape(NW, CH, CHW, F).transpose(0, 3, 1, 2).reshape(NW, ROWS, CHW)
    out = _kernel(xt, table.reshape(1, V))
    return out.reshape(B, 1)
